# baseline (speedup 1.0000x reference)
import functools

import numpy as np

import jax
import jax.numpy as jnp
from jax import lax
from jax.experimental import pallas as pl
from jax.experimental.pallas import tpu as pltpu
from jax.experimental.pallas import tpu_sc as plsc

_NC = 2
_NS = 16
_CA = 128
_CC = 200


def _sc_mesh():
    return plsc.VectorSubcoreMesh(
        core_axis_name="c", subcore_axis_name="s",
        num_cores=_NC, num_subcores=_NS)


_SC_PARAMS = pltpu.CompilerParams(use_tc_tiling_on_sc=False,
                                  needs_layout_passes=False)


def _sc_degrees(ei, maskp, n_pad, sl):
    nw2, nch, cc = ei.shape
    nw = nw2 // 2

    @functools.partial(
        pl.kernel,
        out_type=jax.ShapeDtypeStruct((_NC * n_pad, 128), jnp.float32),
        mesh=_sc_mesh(),
        compiler_params=_SC_PARAMS,
        scratch_types=[
            pltpu.VMEM((nch, cc), jnp.int32),
            pltpu.VMEM((nch, cc), jnp.int32),
            pltpu.VMEM((nch, cc), jnp.float32),
            pltpu.VMEM((sl,), jnp.float32),
            pltpu.VMEM((sl, 128), jnp.float32),
            pltpu.VMEM_SHARED((n_pad,), jnp.float32),
            pltpu.VMEM_SHARED((n_pad,), jnp.float32),
            pltpu.SemaphoreType.DMA,
        ],
    )
    def deg_kernel(ei_hbm, maskp_hbm, out_hbm,
                   idx_s, idx_d, val_v, stage_v, rows_v, dego_sh, degi_sh,
                   sem):
        c = lax.axis_index("c")
        s = lax.axis_index("s")
        w = c * _NS + s
        off = s * sl

        def zbody(k, carry):
            stage_v[pl.ds(k * 16, 16)] = jnp.zeros((16,), jnp.float32)
            return carry
        lax.fori_loop(0, sl // 16, zbody, 0)
        pltpu.sync_copy(stage_v, dego_sh.at[pl.ds(off, sl)])
        pltpu.sync_copy(stage_v, degi_sh.at[pl.ds(off, sl)])
        pltpu.sync_copy(maskp_hbm.at[w], val_v)
        pltpu.sync_copy(ei_hbm.at[w], idx_s)
        pltpu.sync_copy(ei_hbm.at[nw + w], idx_d)
        plsc.subcore_barrier()

        def fire(j, carry):
            pltpu.async_copy(val_v.at[j], dego_sh.at[idx_s.at[j]], sem,
                             add=True)
            pltpu.async_copy(val_v.at[j], degi_sh.at[idx_d.at[j]], sem,
                             add=True)
            return carry
        lax.fori_loop(0, nch, fire, 0)

        def drain(j, carry):
            pltpu.make_async_copy(
                val_v.at[0], dego_sh.at[idx_s.at[0]], sem).wait()
            pltpu.make_async_copy(
                val_v.at[0], degi_sh.at[idx_d.at[0]], sem).wait()
            return carry
        lax.fori_loop(0, nch, drain, 0)

        plsc.subcore_barrier()
        lanes = lax.iota(jnp.int32, 16)

        def pack(col, src_sh):
            pltpu.sync_copy(src_sh.at[pl.ds(off, sl)], stage_v)

            def pbody(r, carry):
                v = stage_v[pl.ds(r * 16, 16)]
                plsc.store_scatter(
                    rows_v, [r * 16 + lanes, jnp.full((16,), col, jnp.int32)],
                    v)
                return carry
            lax.fori_loop(0, sl // 16, pbody, 0)

        pack(0, dego_sh)
        pack(1, degi_sh)
        pltpu.sync_copy(rows_v, out_hbm.at[pl.ds(c * n_pad + off, sl)])

    return deg_kernel(ei, maskp)


def _sc_aggregate(ei, hw, zrow, n_pad, sl, dh):
    nw2, nch, cc = ei.shape
    nw = nw2 // 2
    wl = hw.shape[1]

    @functools.partial(
        pl.kernel,
        out_type=[jax.ShapeDtypeStruct((n_pad, wl), jnp.float32),
                  jax.ShapeDtypeStruct((n_pad, wl), jnp.float32)],
        mesh=_sc_mesh(),
        compiler_params=_SC_PARAMS,
        scratch_types=[
            pltpu.VMEM((nch, cc), jnp.int32),
            pltpu.VMEM((nch, cc), jnp.int32),
            pltpu.VMEM((cc, dh), jnp.float32),
            pltpu.VMEM((cc, dh), jnp.float32),
            pltpu.VMEM_SHARED((n_pad, dh), jnp.float32),
            pltpu.VMEM_SHARED((n_pad, dh), jnp.float32),
            pltpu.SemaphoreType.DMA,
            pltpu.SemaphoreType.DMA,
            pltpu.SemaphoreType.DMA,
            pltpu.SemaphoreType.DMA,
        ],
    )
    def agg_kernel(ei_hbm, hw_hbm, z_hbm, out0_hbm, out1_hbm,
                   src_v, dst_v, buf_a, buf_b, tab_sh, agg_sh,
                   sem_ga, sem_gb, sem_sa, sem_sb):
        c = lax.axis_index("c")
        s = lax.axis_index("s")
        w = c * _NS + s
        off = s * sl

        pltpu.sync_copy(ei_hbm.at[w], src_v)
        pltpu.sync_copy(ei_hbm.at[nw + w], dst_v)
        pltpu.sync_copy(z_hbm, agg_sh.at[pl.ds(off, sl)])
        pltpu.sync_copy(hw_hbm.at[pl.ds(off, sl), pl.ds(0, dh)],
                        tab_sh.at[pl.ds(off, sl)])
        plsc.subcore_barrier()

        def fire_g(j, buf, sem):
            pltpu.async_copy(tab_sh.at[src_v.at[j]], buf, sem)

        def wait_g(buf, sem):
            pltpu.make_async_copy(tab_sh.at[src_v.at[0]], buf, sem).wait()

        def fire_s(j, buf, sem):
            pltpu.async_copy(buf, agg_sh.at[dst_v.at[j]], sem, add=True)

        def wait_s(buf, sem):
            pltpu.make_async_copy(buf, agg_sh.at[dst_v.at[0]], sem).wait()

        fire_g(0, buf_a, sem_ga)
        fire_g(1, buf_b, sem_gb)

        def body(t, carry):
            j = 2 * t
            wait_g(buf_a, sem_ga)
            fire_s(j, buf_a, sem_sa)
            wait_g(buf_b, sem_gb)
            fire_s(j + 1, buf_b, sem_sb)
            wait_s(buf_a, sem_sa)

            @pl.when(t < nch // 2 - 1)
            def _():
                fire_g(j + 2, buf_a, sem_ga)

            wait_s(buf_b, sem_sb)

            @pl.when(t < nch // 2 - 1)
            def _():
                fire_g(j + 3, buf_b, sem_gb)
            return carry
        lax.fori_loop(0, nch // 2, body, 0)

        plsc.subcore_barrier()

        @pl.when(c == 0)
        def _():
            pltpu.sync_copy(agg_sh.at[pl.ds(off, sl)],
                            out0_hbm.at[pl.ds(off, sl), pl.ds(0, dh)])

        @pl.when(c == 1)
        def _():
            pltpu.sync_copy(agg_sh.at[pl.ds(off, sl)],
                            out1_hbm.at[pl.ds(off, sl), pl.ds(0, dh)])

    return agg_kernel(ei, hw, zrow)


def _tc_project(feats, degp, w_mat, n_pad, bn):
    n, di = feats.shape
    wl = w_mat.shape[1]
    nb = n_pad // bn

    def body(feats_ref, d0_ref, d1_ref, w_ref, hw_ref, nd_ref):
        deg_o = d0_ref[:, 0:1] + d1_ref[:, 0:1]
        deg_i = d0_ref[:, 1:2] + d1_ref[:, 1:2]
        norm_o = lax.rsqrt(jnp.maximum(deg_o, 1.0))
        nd_ref[...] = lax.rsqrt(jnp.maximum(deg_i, 1.0))
        h = feats_ref[...] * norm_o
        hw_ref[...] = jnp.dot(h, w_ref[...],
                              preferred_element_type=jnp.float32)

    return pl.pallas_call(
        body,
        grid=(nb,),
        in_specs=[
            pl.BlockSpec((bn, di), lambda i: (i, 0)),
            pl.BlockSpec((bn, 128), lambda i: (i, 0)),
            pl.BlockSpec((bn, 128), lambda i: (i + nb, 0)),
            pl.BlockSpec((di, wl), lambda i: (0, 0)),
        ],
        out_specs=[
            pl.BlockSpec((bn, wl), lambda i: (i, 0)),
            pl.BlockSpec((bn, 1), lambda i: (i, 0)),
        ],
        out_shape=[
            jax.ShapeDtypeStruct((n_pad, wl), jnp.float32),
            jax.ShapeDtypeStruct((n_pad, 1), jnp.float32),
        ],
    )(feats, degp, degp, w_mat)


def _tc_finish(agg0, agg1, norm_dst, b2, n, bn, dh):
    wl = agg0.shape[1]

    n_pad = agg0.shape[0]

    def body(a0_ref, a1_ref, nd_ref, b_ref, out_ref):
        acc = (a0_ref[:, 0:dh] + a1_ref[:, 0:dh]) * nd_ref[...]
        out_ref[...] = jnp.maximum(acc + b_ref[...], 0.0).T

    return pl.pallas_call(
        body,
        grid=(n_pad // bn,),
        in_specs=[
            pl.BlockSpec((bn, wl), lambda i: (i, 0)),
            pl.BlockSpec((bn, wl), lambda i: (i, 0)),
            pl.BlockSpec((bn, 1), lambda i: (i, 0)),
            pl.BlockSpec((1, dh), lambda i: (0, 0)),
        ],
        out_specs=pl.BlockSpec((dh, bn), lambda i: (0, i)),
        out_shape=jax.ShapeDtypeStruct((dh, n), jnp.float32),
    )(agg0, agg1, norm_dst, b2)


def kernel(feats, edge_index, W, b):
    n, di = feats.shape
    dh = W.shape[1]
    e = edge_index.shape[1]
    nw = _NC * _NS

    ept = -(-e // (nw * 2 * _CC)) * 2 * _CC
    pad = nw * ept - e
    sl = -(-(n + 1) // (_NS * _CA)) * _CA
    n_pad = _NS * sl

    if pad:
        srcf = jnp.concatenate([edge_index[0], jnp.zeros((pad,), jnp.int32)])
        dstf = jnp.concatenate([edge_index[1], jnp.full((pad,), n, jnp.int32)])
        ei = jnp.concatenate([srcf, dstf]).reshape(2 * nw, ept // _CC, _CC)
        maskf = jnp.concatenate(
            [jnp.ones((e,), jnp.float32), jnp.zeros((pad,), jnp.float32)])
        maskp = maskf.reshape(nw, ept // _CC, _CC)
    else:
        ei = edge_index.reshape(2 * nw, ept // _CC, _CC)
        maskp = jnp.asarray(
            np.ones((nw, ept // _CC, _CC), np.float32))

    degp = _sc_degrees(ei, maskp, n_pad, sl)

    bn1 = 2048 if n_pad % 2048 == 0 else sl
    w_pad = jnp.pad(W, ((0, 0), (0, 128 - dh))) if dh < 128 else W
    hw, norm_dst = _tc_project(feats, degp, w_pad, n_pad, bn1)

    zrow = jnp.asarray(np.zeros((sl, dh), np.float32))
    agg0, agg1 = _sc_aggregate(ei, hw, zrow, n_pad, sl, dh)

    out_t = _tc_finish(agg0, agg1, norm_dst, b.reshape(1, dh), n, bn1, dh)
    return out_t.T


# speedup vs baseline: 1.0116x; 1.0116x over previous
import functools

import numpy as np

import jax
import jax.numpy as jnp
from jax import lax
from jax.experimental import pallas as pl
from jax.experimental.pallas import tpu as pltpu
from jax.experimental.pallas import tpu_sc as plsc

_NC = 2
_NS = 16
_CA = 128
_CC = 200


def _sc_mesh():
    return plsc.VectorSubcoreMesh(
        core_axis_name="c", subcore_axis_name="s",
        num_cores=_NC, num_subcores=_NS)


_SC_PARAMS = pltpu.CompilerParams(use_tc_tiling_on_sc=False)


def _sc_degrees(ei, maskp, n_pad, sl):
    nw2, nch, cc = ei.shape
    nw = nw2 // 2

    @functools.partial(
        pl.kernel,
        out_type=jax.ShapeDtypeStruct((_NC * 2 * n_pad,), jnp.float32),
        mesh=_sc_mesh(),
        compiler_params=_SC_PARAMS,
        scratch_types=[
            pltpu.VMEM((nch, cc), jnp.int32),
            pltpu.VMEM((nch, cc), jnp.int32),
            pltpu.VMEM((nch, cc), jnp.float32),
            pltpu.VMEM((sl,), jnp.float32),
            pltpu.VMEM_SHARED((n_pad,), jnp.float32),
            pltpu.VMEM_SHARED((n_pad,), jnp.float32),
            pltpu.SemaphoreType.DMA,
        ],
    )
    def deg_kernel(ei_hbm, maskp_hbm, out_hbm,
                   idx_s, idx_d, val_v, stage_v, dego_sh, degi_sh, sem):
        c = lax.axis_index("c")
        s = lax.axis_index("s")
        w = c * _NS + s
        off = s * sl

        def zbody(k, carry):
            stage_v[pl.ds(k * 16, 16)] = jnp.zeros((16,), jnp.float32)
            return carry
        lax.fori_loop(0, sl // 16, zbody, 0)
        pltpu.sync_copy(stage_v, dego_sh.at[pl.ds(off, sl)])
        pltpu.sync_copy(stage_v, degi_sh.at[pl.ds(off, sl)])
        pltpu.sync_copy(maskp_hbm.at[w], val_v)
        pltpu.sync_copy(ei_hbm.at[w], idx_s)
        pltpu.sync_copy(ei_hbm.at[nw + w], idx_d)
        plsc.subcore_barrier()

        def fire(j, carry):
            pltpu.async_copy(val_v.at[j], dego_sh.at[idx_s.at[j]], sem,
                             add=True)
            pltpu.async_copy(val_v.at[j], degi_sh.at[idx_d.at[j]], sem,
                             add=True)
            return carry
        lax.fori_loop(0, nch, fire, 0)

        def drain(j, carry):
            pltpu.make_async_copy(
                val_v.at[0], dego_sh.at[idx_s.at[0]], sem).wait()
            pltpu.make_async_copy(
                val_v.at[0], degi_sh.at[idx_d.at[0]], sem).wait()
            return carry
        lax.fori_loop(0, nch, drain, 0)

        plsc.subcore_barrier()
        pltpu.sync_copy(dego_sh.at[pl.ds(off, sl)], stage_v)
        pltpu.sync_copy(stage_v, out_hbm.at[pl.ds(c * 2 * n_pad + off, sl)])
        pltpu.sync_copy(degi_sh.at[pl.ds(off, sl)], stage_v)
        pltpu.sync_copy(
            stage_v, out_hbm.at[pl.ds((c * 2 + 1) * n_pad + off, sl)])

    return deg_kernel(ei, maskp)


def _sc_aggregate(ei, hw, zrow, n_pad, sl, dh):
    nw2, nch, cc = ei.shape
    nw = nw2 // 2
    wl = hw.shape[1]

    @functools.partial(
        pl.kernel,
        out_type=[jax.ShapeDtypeStruct((n_pad, wl), jnp.float32),
                  jax.ShapeDtypeStruct((n_pad, wl), jnp.float32)],
        mesh=_sc_mesh(),
        compiler_params=_SC_PARAMS,
        scratch_types=[
            pltpu.VMEM((nch, cc), jnp.int32),
            pltpu.VMEM((nch, cc), jnp.int32),
            pltpu.VMEM((cc, dh), jnp.float32),
            pltpu.VMEM((cc, dh), jnp.float32),
            pltpu.VMEM_SHARED((n_pad, dh), jnp.float32),
            pltpu.VMEM_SHARED((n_pad, dh), jnp.float32),
            pltpu.SemaphoreType.DMA,
            pltpu.SemaphoreType.DMA,
            pltpu.SemaphoreType.DMA,
            pltpu.SemaphoreType.DMA,
        ],
    )
    def agg_kernel(ei_hbm, hw_hbm, z_hbm, out0_hbm, out1_hbm,
                   src_v, dst_v, buf_a, buf_b, tab_sh, agg_sh,
                   sem_ga, sem_gb, sem_sa, sem_sb):
        c = lax.axis_index("c")
        s = lax.axis_index("s")
        w = c * _NS + s
        off = s * sl

        pltpu.sync_copy(ei_hbm.at[w], src_v)
        pltpu.sync_copy(ei_hbm.at[nw + w], dst_v)
        pltpu.sync_copy(z_hbm, agg_sh.at[pl.ds(off, sl)])
        pltpu.sync_copy(hw_hbm.at[pl.ds(off, sl), pl.ds(0, dh)],
                        tab_sh.at[pl.ds(off, sl)])
        plsc.subcore_barrier()

        def fire_g(j, buf, sem):
            pltpu.async_copy(tab_sh.at[src_v.at[j]], buf, sem)

        def wait_g(buf, sem):
            pltpu.make_async_copy(tab_sh.at[src_v.at[0]], buf, sem).wait()

        def fire_s(j, buf, sem):
            pltpu.async_copy(buf, agg_sh.at[dst_v.at[j]], sem, add=True)

        def wait_s(buf, sem):
            pltpu.make_async_copy(buf, agg_sh.at[dst_v.at[0]], sem).wait()

        fire_g(0, buf_a, sem_ga)
        fire_g(1, buf_b, sem_gb)

        def body(t, carry):
            j = 2 * t
            wait_g(buf_a, sem_ga)
            fire_s(j, buf_a, sem_sa)
            wait_g(buf_b, sem_gb)
            fire_s(j + 1, buf_b, sem_sb)
            wait_s(buf_a, sem_sa)

            @pl.when(t < nch // 2 - 1)
            def _():
                fire_g(j + 2, buf_a, sem_ga)

            wait_s(buf_b, sem_sb)

            @pl.when(t < nch // 2 - 1)
            def _():
                fire_g(j + 3, buf_b, sem_gb)
            return carry
        lax.fori_loop(0, nch // 2, body, 0)

        plsc.subcore_barrier()

        @pl.when(c == 0)
        def _():
            pltpu.sync_copy(agg_sh.at[pl.ds(off, sl)],
                            out0_hbm.at[pl.ds(off, sl), pl.ds(0, dh)])

        @pl.when(c == 1)
        def _():
            pltpu.sync_copy(agg_sh.at[pl.ds(off, sl)],
                            out1_hbm.at[pl.ds(off, sl), pl.ds(0, dh)])

    return agg_kernel(ei, hw, zrow)


def _tc_project(feats, deg4, w_mat, n_pad, bn):
    n, di = feats.shape
    wl = w_mat.shape[1]

    def body(feats_ref, deg_ref, w_ref, hw_ref, nd_ref):
        deg_o = deg_ref[:, 0:1] + deg_ref[:, 2:3]
        deg_i = deg_ref[:, 1:2] + deg_ref[:, 3:4]
        norm_o = lax.rsqrt(jnp.maximum(deg_o, 1.0))
        nd_ref[...] = lax.rsqrt(jnp.maximum(deg_i, 1.0))
        h = feats_ref[...] * norm_o
        hw_ref[...] = jnp.dot(h, w_ref[...],
                              preferred_element_type=jnp.float32)

    return pl.pallas_call(
        body,
        grid=(n_pad // bn,),
        in_specs=[
            pl.BlockSpec((bn, di), lambda i: (i, 0)),
            pl.BlockSpec((bn, 4), lambda i: (i, 0)),
            pl.BlockSpec((di, wl), lambda i: (0, 0)),
        ],
        out_specs=[
            pl.BlockSpec((bn, wl), lambda i: (i, 0)),
            pl.BlockSpec((bn, 1), lambda i: (i, 0)),
        ],
        out_shape=[
            jax.ShapeDtypeStruct((n_pad, wl), jnp.float32),
            jax.ShapeDtypeStruct((n_pad, 1), jnp.float32),
        ],
    )(feats, deg4, w_mat)


def _tc_finish(agg0, agg1, norm_dst, b2, n, bn, dh):
    wl = agg0.shape[1]

    n_pad = agg0.shape[0]

    def body(a0_ref, a1_ref, nd_ref, b_ref, out_ref):
        acc = (a0_ref[:, 0:dh] + a1_ref[:, 0:dh]) * nd_ref[...]
        out_ref[...] = jnp.maximum(acc + b_ref[...], 0.0).T

    return pl.pallas_call(
        body,
        grid=(n_pad // bn,),
        in_specs=[
            pl.BlockSpec((bn, wl), lambda i: (i, 0)),
            pl.BlockSpec((bn, wl), lambda i: (i, 0)),
            pl.BlockSpec((bn, 1), lambda i: (i, 0)),
            pl.BlockSpec((1, dh), lambda i: (0, 0)),
        ],
        out_specs=pl.BlockSpec((dh, bn), lambda i: (0, i)),
        out_shape=jax.ShapeDtypeStruct((dh, n), jnp.float32),
    )(agg0, agg1, norm_dst, b2)


def kernel(feats, edge_index, W, b):
    n, di = feats.shape
    dh = W.shape[1]
    e = edge_index.shape[1]
    nw = _NC * _NS

    ept = -(-e // (nw * 2 * _CC)) * 2 * _CC
    pad = nw * ept - e
    sl = -(-(n + 1) // (_NS * _CA)) * _CA
    n_pad = _NS * sl

    if pad:
        srcf = jnp.concatenate([edge_index[0], jnp.zeros((pad,), jnp.int32)])
        dstf = jnp.concatenate([edge_index[1], jnp.full((pad,), n, jnp.int32)])
        ei = jnp.concatenate([srcf, dstf]).reshape(2 * nw, ept // _CC, _CC)
        maskf = jnp.concatenate(
            [jnp.ones((e,), jnp.float32), jnp.zeros((pad,), jnp.float32)])
        maskp = maskf.reshape(nw, ept // _CC, _CC)
    else:
        ei = edge_index.reshape(2 * nw, ept // _CC, _CC)
        maskp = jnp.asarray(
            np.ones((nw, ept // _CC, _CC), np.float32))

    deg_part = _sc_degrees(ei, maskp, n_pad, sl).reshape(4, n_pad)
    deg4 = deg_part[:, :n].T

    bn1 = 2048 if n_pad % 2048 == 0 else sl
    w_pad = jnp.pad(W, ((0, 0), (0, 128 - dh))) if dh < 128 else W
    hw, norm_dst = _tc_project(feats, deg4, w_pad, n_pad, bn1)

    zrow = jnp.asarray(np.zeros((sl, dh), np.float32))
    agg0, agg1 = _sc_aggregate(ei, hw, zrow, n_pad, sl, dh)

    out_t = _tc_finish(agg0, agg1, norm_dst, b.reshape(1, dh), n, bn1, dh)
    return out_t.T
